# Initial kernel scaffold; baseline (speedup 1.0000x reference)
import jax
import jax.numpy as jnp
from jax.experimental import pallas as pl


def kernel(x, spiral_indices, W1, b1, W2, b2):
    raise NotImplementedError("write your pallas kernel here")


# speedup vs baseline: 2.9718x; 2.9718x over previous
import functools

import jax
import jax.numpy as jnp
from jax import lax
from jax.experimental import pallas as pl
from jax.experimental.pallas import tpu as pltpu
from jax.experimental.pallas import tpu_sc as plsc


def _mm(x, w, bn):
    n, k = x.shape
    m = w.shape[1]
    nb = n // bn

    def body(x_ref, w_ref, o_ref):
        o_ref[...] = jnp.dot(x_ref[...], w_ref[...],
                             preferred_element_type=jnp.float32)

    return pl.pallas_call(
        body,
        grid=(nb,),
        in_specs=[pl.BlockSpec((bn, k), lambda i: (i, 0)),
                  pl.BlockSpec((k, m), lambda i: (0, 0))],
        out_specs=pl.BlockSpec((bn, m), lambda i: (i, 0)),
        out_shape=jax.ShapeDtypeStruct((n, m), jnp.float32),
    )(x, w)


def _make_gather_sum(np_, seq, c, ch, elu):
    info = plsc.get_sparse_core_info()
    nc, ns = info.num_cores, info.num_subcores
    nw = nc * ns
    npw = np_ // nw
    nch = npw // ch
    g = ch * seq
    nj = c // 16
    kp = 128 // c
    nrow = np_ * seq // kp
    mesh = plsc.VectorSubcoreMesh(core_axis_name="c", subcore_axis_name="s")

    @functools.partial(
        pl.kernel, mesh=mesh,
        out_type=jax.ShapeDtypeStruct((np_, c), jnp.float32),
        scratch_types=[
            pltpu.VMEM((g,), jnp.int32),
            pltpu.VMEM((g, 128), jnp.float32),
            pltpu.VMEM((ch, c), jnp.float32),
            pltpu.VMEM((c,), jnp.float32),
            pltpu.SemaphoreType.DMA,
        ],
    )
    def k(table_hbm, idx_hbm, b_hbm, out_hbm, idx_v, rows_v, hv, b_v, sem):
        wid = lax.axis_index("s") * nc + lax.axis_index("c")
        base_node = wid * npw
        pltpu.sync_copy(b_hbm, b_v)
        bias = [b_v[pl.ds(16 * j, 16)] for j in range(nj)]

        def chunk_body(cidx, carry):
            nb0 = base_node + cidx * ch
            pltpu.sync_copy(idx_hbm.at[pl.ds(nb0 * seq, g)], idx_v)
            pltpu.async_copy(table_hbm.at[idx_v], rows_v, sem).wait()

            def node_body(i, carry2):
                r0 = i * seq
                for j in range(nj):
                    acc = rows_v[r0, pl.ds(16 * j, 16)]
                    for s in range(1, seq):
                        off = (s % kp) * c + 16 * j
                        acc = acc + rows_v[r0 + s, pl.ds(off, 16)]
                    acc = acc + bias[j]
                    if elu:
                        acc = jnp.where(
                            acc > 0.0,
                            acc,
                            jnp.exp(jnp.minimum(acc, 0.0)) - 1.0)
                    hv[i, pl.ds(16 * j, 16)] = acc
                return carry2

            lax.fori_loop(0, ch, node_body, 0)
            pltpu.sync_copy(hv, out_hbm.at[pl.ds(nb0, ch)])
            return carry

        lax.fori_loop(0, nch, chunk_body, 0)

    return k


def kernel(x, spiral_indices, W1, b1, W2, b2):
    n = x.shape[0]
    seq = spiral_indices.shape[1]
    xs = jnp.squeeze(x, -1)
    d0 = xs.shape[1]
    d1 = W1.shape[1]
    d2 = W2.shape[1]

    np_ = ((n + 2047) // 2048) * 2048
    xp = jnp.pad(xs, ((0, np_ - n), (0, 0)))
    pad_idx = ((jnp.arange(np_ - n, dtype=jnp.int32)[:, None]
                + jnp.arange(seq, dtype=jnp.int32)[None, :]) * 97) % n
    idxp = jnp.concatenate([spiral_indices, pad_idx], axis=0)

    wc1 = W1.reshape(seq, d0, d1).transpose(1, 0, 2).reshape(d0, seq * d1)
    wc2 = W2.reshape(seq, d1, d2).transpose(1, 0, 2).reshape(d1, seq * d2)

    kp1 = 128 // d1
    kp2 = 128 // d2
    s_ar = jnp.arange(seq, dtype=jnp.int32)[None, :]
    idx2a = (idxp * (seq // kp1) + s_ar // kp1).reshape(-1)
    idx2b = (idxp * (seq // kp2) + s_ar // kp2).reshape(-1)

    p1 = _mm(xp, wc1, 2048).reshape(np_ * seq // kp1, 128)
    h = _make_gather_sum(np_, seq, d1, 32, True)(p1, idx2a, b1)
    p2 = _mm(h, wc2, 2048).reshape(np_ * seq // kp2, 128)
    o = _make_gather_sum(np_, seq, d2, 32, False)(p2, idx2b, b2)
    return o[:n, :, None]


# speedup vs baseline: 3.7881x; 1.2747x over previous
import functools

import jax
import jax.numpy as jnp
from jax import lax
from jax.experimental import pallas as pl
from jax.experimental.pallas import tpu as pltpu
from jax.experimental.pallas import tpu_sc as plsc


def _mm(x, w, bn):
    n, k = x.shape
    m = w.shape[1]
    nb = n // bn

    def body(x_ref, w_ref, o_ref):
        o_ref[...] = jnp.dot(x_ref[...], w_ref[...],
                             preferred_element_type=jnp.float32)

    return pl.pallas_call(
        body,
        grid=(nb,),
        in_specs=[pl.BlockSpec((bn, k), lambda i: (i, 0)),
                  pl.BlockSpec((k, m), lambda i: (0, 0))],
        out_specs=pl.BlockSpec((bn, m), lambda i: (i, 0)),
        out_shape=jax.ShapeDtypeStruct((n, m), jnp.float32),
    )(x, w)


def _make_gather_sum(np_, seq, c, ch, elu):
    info = plsc.get_sparse_core_info()
    nc, ns = info.num_cores, info.num_subcores
    nw = nc * ns
    npw = np_ // nw
    nch = npw // ch
    g = ch * seq
    nj = c // 16
    kp = 128 // c
    mesh = plsc.VectorSubcoreMesh(core_axis_name="c", subcore_axis_name="s")

    @functools.partial(
        pl.kernel, mesh=mesh,
        out_type=jax.ShapeDtypeStruct((np_, c), jnp.float32),
        scratch_types=[
            pltpu.VMEM((npw * seq,), jnp.int32),
            pltpu.VMEM((g, 128), jnp.float32),
            pltpu.VMEM((g, 128), jnp.float32),
            pltpu.VMEM((ch, c), jnp.float32),
            pltpu.VMEM((c,), jnp.float32),
            pltpu.SemaphoreType.DMA,
            pltpu.SemaphoreType.DMA,
        ],
    )
    def k(table_hbm, idx_hbm, b_hbm, out_hbm,
          idx_v, rows0, rows1, hv, b_v, sem0, sem1):
        wid = lax.axis_index("s") * nc + lax.axis_index("c")
        base_node = wid * npw
        pltpu.sync_copy(idx_hbm.at[pl.ds(base_node * seq, npw * seq)], idx_v)
        pltpu.sync_copy(b_hbm, b_v)
        bias = [b_v[pl.ds(16 * j, 16)] for j in range(nj)]

        def fire(cidx, rows_v, sem):
            pltpu.async_copy(
                table_hbm.at[idx_v.at[pl.ds(cidx * g, g)]], rows_v, sem)

        def drain(cidx, rows_v, sem):
            pltpu.make_async_copy(
                table_hbm.at[idx_v.at[pl.ds(cidx * g, g)]], rows_v, sem
            ).wait()

        def compute(cidx, rows_v):
            def node_body(i, carry2):
                r0 = i * seq
                for j in range(nj):
                    acc = rows_v[r0, pl.ds(16 * j, 16)]
                    for s in range(1, seq):
                        off = (s % kp) * c + 16 * j
                        acc = acc + rows_v[r0 + s, pl.ds(off, 16)]
                    acc = acc + bias[j]
                    if elu:
                        acc = jnp.where(
                            acc > 0.0,
                            acc,
                            jnp.exp(jnp.minimum(acc, 0.0)) - 1.0)
                    hv[i, pl.ds(16 * j, 16)] = acc
                return carry2

            lax.fori_loop(0, ch, node_body, 0)
            pltpu.sync_copy(hv, out_hbm.at[pl.ds(base_node + cidx * ch, ch)])

        fire(0, rows0, sem0)

        def pair_body(p, carry):
            c0 = 2 * p
            fire(c0 + 1, rows1, sem1)
            drain(c0, rows0, sem0)
            compute(c0, rows0)
            fire(jnp.minimum(c0 + 2, nch - 1), rows0, sem0)
            drain(c0 + 1, rows1, sem1)
            compute(c0 + 1, rows1)
            return carry

        lax.fori_loop(0, nch // 2, pair_body, 0)
        drain(nch - 1, rows0, sem0)

    return k


def kernel(x, spiral_indices, W1, b1, W2, b2):
    n = x.shape[0]
    seq = spiral_indices.shape[1]
    xs = jnp.squeeze(x, -1)
    d0 = xs.shape[1]
    d1 = W1.shape[1]
    d2 = W2.shape[1]

    np_ = ((n + 2047) // 2048) * 2048
    xp = jnp.pad(xs, ((0, np_ - n), (0, 0)))
    pad_idx = ((jnp.arange(np_ - n, dtype=jnp.int32)[:, None]
                + jnp.arange(seq, dtype=jnp.int32)[None, :]) * 97) % n
    idxp = jnp.concatenate([spiral_indices, pad_idx], axis=0)

    wc1 = W1.reshape(seq, d0, d1).transpose(1, 0, 2).reshape(d0, seq * d1)
    wc2 = W2.reshape(seq, d1, d2).transpose(1, 0, 2).reshape(d1, seq * d2)

    kp1 = 128 // d1
    kp2 = 128 // d2
    s_ar = jnp.arange(seq, dtype=jnp.int32)[None, :]
    idx2a = (idxp * (seq // kp1) + s_ar // kp1).reshape(-1)
    idx2b = (idxp * (seq // kp2) + s_ar // kp2).reshape(-1)

    p1 = _mm(xp, wc1, 2048).reshape(np_ * seq // kp1, 128)
    h = _make_gather_sum(np_, seq, d1, 16, True)(p1, idx2a, b1)
    p2 = _mm(h, wc2, 2048).reshape(np_ * seq // kp2, 128)
    o = _make_gather_sum(np_, seq, d2, 16, False)(p2, idx2b, b2)
    return o[:n, :, None]


# speedup vs baseline: 4.5858x; 1.2106x over previous
import functools

import jax
import jax.numpy as jnp
from jax import lax
from jax.experimental import pallas as pl
from jax.experimental.pallas import tpu as pltpu
from jax.experimental.pallas import tpu_sc as plsc


def _mm(x, w, bn):
    n, k = x.shape
    m = w.shape[1]
    nb = n // bn

    def body(x_ref, w_ref, o_ref):
        o_ref[...] = jnp.dot(x_ref[...], w_ref[...],
                             preferred_element_type=jnp.float32)

    return pl.pallas_call(
        body,
        grid=(nb,),
        in_specs=[pl.BlockSpec((bn, k), lambda i: (i, 0)),
                  pl.BlockSpec((k, m), lambda i: (0, 0))],
        out_specs=pl.BlockSpec((bn, m), lambda i: (i, 0)),
        out_shape=jax.ShapeDtypeStruct((n, m), jnp.float32),
    )(x, w)


def _make_gather_sum(np_, seq, c, ch, elu):
    info = plsc.get_sparse_core_info()
    nc, ns = info.num_cores, info.num_subcores
    nw = nc * ns
    npw = np_ // nw
    nch = npw // ch
    g = ch * seq
    nj = c // 16
    mesh = plsc.VectorSubcoreMesh(core_axis_name="c", subcore_axis_name="s")

    @functools.partial(
        pl.kernel, mesh=mesh,
        out_type=jax.ShapeDtypeStruct((np_, c), jnp.float32),
        scratch_types=[
            pltpu.VMEM((npw * seq,), jnp.int32),
            pltpu.VMEM((g, c), jnp.float32),
            pltpu.VMEM((g, c), jnp.float32),
            pltpu.VMEM((ch, c), jnp.float32),
            pltpu.VMEM((c,), jnp.float32),
            pltpu.SemaphoreType.DMA,
            pltpu.SemaphoreType.DMA,
        ],
        compiler_params=pltpu.CompilerParams(use_tc_tiling_on_sc=False),
    )
    def k(table_hbm, idx_hbm, b_hbm, out_hbm,
          idx_v, rows0, rows1, hv, b_v, sem0, sem1):
        wid = lax.axis_index("s") * nc + lax.axis_index("c")
        base_node = wid * npw
        pltpu.sync_copy(idx_hbm.at[pl.ds(base_node * seq, npw * seq)], idx_v)
        pltpu.sync_copy(b_hbm, b_v)
        bias = [b_v[pl.ds(16 * j, 16)] for j in range(nj)]

        def fire(cidx, rows_v, sem):
            pltpu.async_copy(
                table_hbm.at[idx_v.at[pl.ds(cidx * g, g)]], rows_v, sem)

        def drain(cidx, rows_v, sem):
            pltpu.make_async_copy(
                table_hbm.at[idx_v.at[pl.ds(cidx * g, g)]], rows_v, sem
            ).wait()

        def compute(cidx, rows_v):
            def node_body(i, carry2):
                r0 = i * seq
                for j in range(nj):
                    acc = rows_v[r0, pl.ds(16 * j, 16)]
                    for s in range(1, seq):
                        acc = acc + rows_v[r0 + s, pl.ds(16 * j, 16)]
                    acc = acc + bias[j]
                    if elu:
                        acc = jnp.where(
                            acc > 0.0,
                            acc,
                            jnp.exp(jnp.minimum(acc, 0.0)) - 1.0)
                    hv[i, pl.ds(16 * j, 16)] = acc
                return carry2

            lax.fori_loop(0, ch, node_body, 0)
            pltpu.sync_copy(hv, out_hbm.at[pl.ds(base_node + cidx * ch, ch)])

        fire(0, rows0, sem0)

        def pair_body(p, carry):
            c0 = 2 * p
            fire(c0 + 1, rows1, sem1)
            drain(c0, rows0, sem0)
            compute(c0, rows0)
            fire(jnp.minimum(c0 + 2, nch - 1), rows0, sem0)
            drain(c0 + 1, rows1, sem1)
            compute(c0 + 1, rows1)
            return carry

        lax.fori_loop(0, nch // 2, pair_body, 0)
        drain(nch - 1, rows0, sem0)

    return k


def kernel(x, spiral_indices, W1, b1, W2, b2):
    n = x.shape[0]
    seq = spiral_indices.shape[1]
    xs = jnp.squeeze(x, -1)
    d0 = xs.shape[1]
    d1 = W1.shape[1]
    d2 = W2.shape[1]

    np_ = ((n + 2047) // 2048) * 2048
    xp = jnp.pad(xs, ((0, np_ - n), (0, 0)))
    pad_idx = ((jnp.arange(np_ - n, dtype=jnp.int32)[:, None]
                + jnp.arange(seq, dtype=jnp.int32)[None, :]) * 97) % n
    idxp = jnp.concatenate([spiral_indices, pad_idx], axis=0)

    wc1 = W1.reshape(seq, d0, d1).transpose(1, 0, 2).reshape(d0, seq * d1)
    wc2 = W2.reshape(seq, d1, d2).transpose(1, 0, 2).reshape(d1, seq * d2)

    s_ar = jnp.arange(seq, dtype=jnp.int32)[None, :]
    idx2 = (idxp * seq + s_ar).reshape(-1)

    p1 = _mm(xp, wc1, 2048).reshape(np_ * seq, d1)
    h = _make_gather_sum(np_, seq, d1, 16, True)(p1, idx2, b1)
    p2 = _mm(h, wc2, 2048).reshape(np_ * seq, d2)
    o = _make_gather_sum(np_, seq, d2, 16, False)(p2, idx2, b2)
    return o[:n, :, None]
